# baseline (speedup 1.0000x reference)
import jax
import jax.numpy as jnp
from jax.experimental import pallas as pl

_B_BLOCK = 256


def _rule_kernel(mf_ref, idx_ref, firing_ref, norm_ref):
    r, f = idx_ref.shape
    k = mf_ref.shape[1]
    m = k // f
    idx = idx_ref[...]
    idx_exp = jnp.broadcast_to(idx[:, :, None], (r, f, m)).reshape(r, k)
    m_of_k = jax.lax.broadcasted_iota(jnp.int32, (1, k), 1) % m
    wt = (idx_exp == m_of_k).astype(jnp.float32)
    logs = jnp.log(mf_ref[...] + 1e-9)
    log_firing = jax.lax.dot_general(
        logs, wt, (((1,), (1,)), ((), ())),
        preferred_element_type=jnp.float32,
        precision=jax.lax.Precision.HIGHEST)
    firing = jnp.exp(log_firing)
    s = jnp.sum(firing, axis=1, keepdims=True) + 1e-6
    firing_ref[...] = firing
    norm_ref[...] = firing / s


def kernel(mf_values, rule_indices):
    b, f, m = mf_values.shape
    r = rule_indices.shape[0]
    k = f * m
    mf_flat = jnp.reshape(mf_values, (b, k))
    idx = rule_indices.astype(jnp.int32)
    nb = b // _B_BLOCK
    firing, norm = pl.pallas_call(
        _rule_kernel,
        grid=(nb,),
        in_specs=[
            pl.BlockSpec((_B_BLOCK, k), lambda i: (i, 0)),
            pl.BlockSpec((r, f), lambda i: (0, 0)),
        ],
        out_specs=(
            pl.BlockSpec((_B_BLOCK, r), lambda i: (i, 0)),
            pl.BlockSpec((_B_BLOCK, r), lambda i: (i, 0)),
        ),
        out_shape=(jax.ShapeDtypeStruct((b, r), jnp.float32),
                   jax.ShapeDtypeStruct((b, r), jnp.float32)),
    )(mf_flat, idx)
    return firing, norm


# speedup vs baseline: 1.8855x; 1.8855x over previous
import jax
import jax.numpy as jnp
from jax.experimental import pallas as pl


def _rule_kernel(mf_ref, idx_ref, firing_ref, norm_ref):
    r, f = idx_ref.shape
    k = mf_ref.shape[1]
    m = k // f
    idx = idx_ref[...]
    idx_exp = jnp.broadcast_to(idx[:, :, None], (r, f, m)).reshape(r, k)
    m_of_k = jax.lax.broadcasted_iota(jnp.int32, (1, k), 1) % m
    wt = (idx_exp == m_of_k).astype(jnp.float32)
    logs = jnp.log(mf_ref[...] + 1e-9)
    log_firing = jax.lax.dot_general(
        logs, wt, (((1,), (1,)), ((), ())),
        preferred_element_type=jnp.float32,
        precision=jax.lax.Precision.HIGHEST)
    firing = jnp.exp(log_firing)
    s = jnp.sum(firing, axis=1, keepdims=True) + 1e-6
    firing_ref[...] = firing
    norm_ref[...] = firing / s


def kernel(mf_values, rule_indices):
    b, f, m = mf_values.shape
    r = rule_indices.shape[0]
    k = f * m
    mf_flat = jnp.reshape(mf_values, (b, k))
    idx = rule_indices.astype(jnp.int32)
    firing, norm = pl.pallas_call(
        _rule_kernel,
        out_shape=(jax.ShapeDtypeStruct((b, r), jnp.float32),
                   jax.ShapeDtypeStruct((b, r), jnp.float32)),
    )(mf_flat, idx)
    return firing, norm


# speedup vs baseline: 2.6362x; 1.3981x over previous
import jax
import jax.numpy as jnp
from jax.experimental import pallas as pl


def _rule_kernel(mf_ref, idx_ref, firing_ref, norm_ref):
    r, f = idx_ref.shape
    k = mf_ref.shape[1]
    m = k // f
    idxt = idx_ref[...].T
    idx_exp = jnp.broadcast_to(idxt[:, None, :], (f, m, r)).reshape(k, r)
    m_of_k = jax.lax.broadcasted_iota(jnp.int32, (k, 1), 0) % m
    w = (idx_exp == m_of_k).astype(jnp.float32)
    logs = jnp.log(mf_ref[...] + 1e-9)
    log_firing = jax.lax.dot_general(
        logs, w, (((1,), (0,)), ((), ())),
        preferred_element_type=jnp.float32,
        precision=jax.lax.Precision.HIGHEST)
    firing = jnp.exp(log_firing)
    s = jnp.sum(firing, axis=1, keepdims=True) + 1e-6
    firing_ref[...] = firing
    norm_ref[...] = firing / s


def kernel(mf_values, rule_indices):
    b, f, m = mf_values.shape
    r = rule_indices.shape[0]
    mf_flat = jnp.reshape(mf_values, (b, f * m))
    idx = rule_indices.astype(jnp.int32)
    firing, norm = pl.pallas_call(
        _rule_kernel,
        out_shape=(jax.ShapeDtypeStruct((b, r), jnp.float32),
                   jax.ShapeDtypeStruct((b, r), jnp.float32)),
    )(mf_flat, idx)
    return firing, norm


# speedup vs baseline: 3.1679x; 1.2017x over previous
import jax
import jax.numpy as jnp
from jax.experimental import pallas as pl


def _rule_kernel(mf_ref, idxt_ref, firing_ref, norm_ref):
    f, r = idxt_ref.shape
    k = mf_ref.shape[1]
    m = k // f
    idxt = idxt_ref[...]
    idx_exp = jnp.broadcast_to(idxt[:, None, :], (f, m, r)).reshape(k, r)
    m_of_k = jax.lax.broadcasted_iota(jnp.int32, (k, 1), 0) % m
    w = (idx_exp == m_of_k).astype(jnp.float32)
    logs = jnp.log(mf_ref[...] + 1e-9)
    log_firing = jax.lax.dot_general(
        logs, w, (((1,), (0,)), ((), ())),
        preferred_element_type=jnp.float32,
        precision=jax.lax.Precision.HIGHEST)
    firing = jnp.exp(log_firing)
    s = jnp.sum(firing, axis=1, keepdims=True) + 1e-6
    firing_ref[...] = firing
    norm_ref[...] = firing / s


def kernel(mf_values, rule_indices):
    b, f, m = mf_values.shape
    r = rule_indices.shape[0]
    mf_flat = jnp.reshape(mf_values, (b, f * m))
    idxt = rule_indices.astype(jnp.int32).T
    firing, norm = pl.pallas_call(
        _rule_kernel,
        out_shape=(jax.ShapeDtypeStruct((b, r), jnp.float32),
                   jax.ShapeDtypeStruct((b, r), jnp.float32)),
    )(mf_flat, idxt)
    return firing, norm
